# Initial kernel scaffold; baseline (speedup 1.0000x reference)
import jax
import jax.numpy as jnp
from jax.experimental import pallas as pl


def kernel(input, gate_weight):
    raise NotImplementedError("write your pallas kernel here")


# speedup vs baseline: 1.4160x; 1.4160x over previous
import functools

import jax
import jax.numpy as jnp
from jax.experimental import pallas as pl
from jax.experimental.pallas import tpu as pltpu

N_TOKENS = 16384
D_MODEL = 2048
N_EXPERTS = 16
TOP_K = 2

BT = 1024


def _router_block(x_ref, w_ref, probs_ref, idx_ref, hist_ref):
    x = x_ref[...]
    w = w_ref[...]
    logits = jax.lax.dot_general(
        x, w, (((1,), (1,)), ((), ())), preferred_element_type=jnp.float32
    )

    e_ids = jax.lax.broadcasted_iota(jnp.int32, (BT, N_EXPERTS), 1)
    m1 = jnp.max(logits, axis=1, keepdims=True)
    i1 = jnp.min(jnp.where(logits == m1, e_ids, N_EXPERTS), axis=1, keepdims=True)
    masked = jnp.where(e_ids == i1, -jnp.inf, logits)
    m2 = jnp.max(masked, axis=1, keepdims=True)
    i2 = jnp.min(jnp.where(masked == m2, e_ids, N_EXPERTS), axis=1, keepdims=True)

    e = jnp.exp(m2 - m1)
    s = 1.0 / (1.0 + e)
    probs_ref[...] = jnp.concatenate([s, e * s], axis=1)
    idx_ref[...] = jnp.concatenate([i1, i2], axis=1)

    counts = jnp.sum(
        (e_ids == i1).astype(jnp.int32) + (e_ids == i2).astype(jnp.int32),
        axis=0,
        keepdims=True,
    )

    @pl.when(pl.program_id(0) == 0)
    def _():
        hist_ref[...] = jnp.zeros_like(hist_ref)

    hist_ref[...] += counts


@functools.partial(jax.jit, static_argnames=())
def _run(x, w):
    grid = N_TOKENS // BT
    probs, idx, hist = pl.pallas_call(
        _router_block,
        grid=(grid,),
        in_specs=[
            pl.BlockSpec((BT, D_MODEL), lambda i: (i, 0)),
            pl.BlockSpec((N_EXPERTS, D_MODEL), lambda i: (0, 0)),
        ],
        out_specs=[
            pl.BlockSpec((BT, TOP_K), lambda i: (i, 0)),
            pl.BlockSpec((BT, TOP_K), lambda i: (i, 0)),
            pl.BlockSpec((1, N_EXPERTS), lambda i: (0, 0)),
        ],
        out_shape=[
            jax.ShapeDtypeStruct((N_TOKENS, TOP_K), jnp.float32),
            jax.ShapeDtypeStruct((N_TOKENS, TOP_K), jnp.int32),
            jax.ShapeDtypeStruct((1, N_EXPERTS), jnp.int32),
        ],
        compiler_params=pltpu.CompilerParams(
            dimension_semantics=("arbitrary",),
        ),
    )(x, w)
    return probs, idx, hist.reshape(N_EXPERTS)


def kernel(input, gate_weight):
    return _run(input, gate_weight)


# speedup vs baseline: 1.4732x; 1.0404x over previous
import functools

import jax
import jax.numpy as jnp
from jax.experimental import pallas as pl
from jax.experimental.pallas import tpu as pltpu

N_TOKENS = 16384
D_MODEL = 2048
N_EXPERTS = 16
TOP_K = 2

BT = 2048


def _router_block(x_ref, w_ref, probs_ref, idx_ref, hist_ref):
    x = x_ref[...]
    w = w_ref[...]
    logits = jax.lax.dot_general(
        x, w, (((1,), (1,)), ((), ())), preferred_element_type=jnp.float32
    )

    e_ids = jax.lax.broadcasted_iota(jnp.int32, (BT, N_EXPERTS), 1)
    m1 = jnp.max(logits, axis=1, keepdims=True)
    i1 = jnp.min(jnp.where(logits == m1, e_ids, N_EXPERTS), axis=1, keepdims=True)
    masked = jnp.where(e_ids == i1, -jnp.inf, logits)
    m2 = jnp.max(masked, axis=1, keepdims=True)
    i2 = jnp.min(jnp.where(masked == m2, e_ids, N_EXPERTS), axis=1, keepdims=True)

    e = jnp.exp(m2 - m1)
    s = 1.0 / (1.0 + e)
    probs_ref[...] = jnp.concatenate([s, e * s], axis=1)
    idx_ref[...] = jnp.concatenate([i1, i2], axis=1)

    counts = jnp.sum(
        (e_ids == i1).astype(jnp.int32) + (e_ids == i2).astype(jnp.int32),
        axis=0,
        keepdims=True,
    )

    @pl.when(pl.program_id(0) == 0)
    def _():
        hist_ref[...] = jnp.zeros_like(hist_ref)

    hist_ref[...] += counts


@functools.partial(jax.jit, static_argnames=())
def _run(x, w):
    grid = N_TOKENS // BT
    probs, idx, hist = pl.pallas_call(
        _router_block,
        grid=(grid,),
        in_specs=[
            pl.BlockSpec((BT, D_MODEL), lambda i: (i, 0)),
            pl.BlockSpec((N_EXPERTS, D_MODEL), lambda i: (0, 0)),
        ],
        out_specs=[
            pl.BlockSpec((BT, TOP_K), lambda i: (i, 0)),
            pl.BlockSpec((BT, TOP_K), lambda i: (i, 0)),
            pl.BlockSpec((1, N_EXPERTS), lambda i: (0, 0)),
        ],
        out_shape=[
            jax.ShapeDtypeStruct((N_TOKENS, TOP_K), jnp.float32),
            jax.ShapeDtypeStruct((N_TOKENS, TOP_K), jnp.int32),
            jax.ShapeDtypeStruct((1, N_EXPERTS), jnp.int32),
        ],
        compiler_params=pltpu.CompilerParams(
            dimension_semantics=("arbitrary",),
        ),
    )(x, w)
    return probs, idx, hist.reshape(N_EXPERTS)


def kernel(input, gate_weight):
    return _run(input, gate_weight)
